# baseline (speedup 1.0000x reference)
import functools

import jax
import jax.numpy as jnp
from jax import lax
from jax.experimental import pallas as pl
from jax.experimental.pallas import tpu as pltpu
from jax.experimental.pallas import tpu_sc as plsc

N = 10000
E = 320000
D_IN = 128
H1 = 64
H2 = 16

NC = 2
NS = 16
NW = NC * NS
CHUNK = 250
NCH = E // (NW * CHUNK)
NPAD = 10240
ROWS_PER_TILE = NPAD // NS
ZR = 64


def _spmm_sc(sup, src2d, dst2d, D, NBUF):
  NG = NCH // NBUF
  mesh = plsc.VectorSubcoreMesh(core_axis_name="c", subcore_axis_name="s")

  def body(sup_hbm, src_hbm, dst_hbm, out_hbm,
           src_v, dst_v, rows_v, zblk_v, acc_sh,
           sem_g0, sem_g1, sem_s0, sem_s1):
    cid = lax.axis_index("c")
    sid = lax.axis_index("s")
    w = sid * NC + cid
    sem_g = (sem_g0, sem_g1)
    sem_s = (sem_s0, sem_s1)

    zero16 = jnp.zeros((16,), jnp.float32)

    def zrow(i, carry):
      for c in range(D // 16):
        zblk_v[i, pl.ds(16 * c, 16)] = zero16
      return carry

    lax.fori_loop(0, ZR, zrow, 0)
    for j in range(ROWS_PER_TILE // ZR):
      pltpu.sync_copy(zblk_v,
                      acc_sh.at[pl.ds(sid * ROWS_PER_TILE + j * ZR, ZR), :])
    plsc.subcore_barrier()

    pltpu.sync_copy(src_hbm.at[pl.ds(w * NCH, NCH), :], src_v)
    pltpu.sync_copy(dst_hbm.at[pl.ds(w * NCH, NCH), :], dst_v)

    def fire_g(grp, g):
      for b in range(NBUF):
        pltpu.async_copy(sup_hbm.at[src_v.at[g * NBUF + b]],
                         rows_v.at[grp * NBUF + b], sem_g[grp])

    def drain_g(grp):
      for b in range(NBUF):
        pltpu.make_async_copy(sup_hbm.at[src_v.at[0]],
                              rows_v.at[grp * NBUF + b], sem_g[grp]).wait()

    def fire_s(grp, g):
      for b in range(NBUF):
        pltpu.async_copy(rows_v.at[grp * NBUF + b],
                         acc_sh.at[dst_v.at[g * NBUF + b]], sem_s[grp],
                         add=True)

    def drain_s(grp):
      for b in range(NBUF):
        pltpu.make_async_copy(rows_v.at[grp * NBUF + b],
                              acc_sh.at[dst_v.at[0]], sem_s[grp]).wait()

    fire_g(0, 0)

    def itr(k, carry):
      g0 = 2 * k
      g1 = 2 * k + 1
      drain_g(0)

      @pl.when(k > 0)
      def _():
        drain_s(1)

      fire_g(1, g1)
      fire_s(0, g0)
      drain_g(1)
      drain_s(0)

      @pl.when(k < NG // 2 - 1)
      def _():
        fire_g(0, g0 + 2)

      fire_s(1, g1)
      return carry

    lax.fori_loop(0, NG // 2, itr, 0)
    drain_s(1)
    plsc.subcore_barrier()

    for j in range(ROWS_PER_TILE // ZR):
      r0 = sid * ROWS_PER_TILE + j * ZR
      pltpu.sync_copy(acc_sh.at[pl.ds(r0, ZR), :],
                      out_hbm.at[cid, pl.ds(r0, ZR), :])

  f = pl.kernel(
      body,
      out_type=jax.ShapeDtypeStruct((NC, NPAD, D), jnp.float32),
      mesh=mesh,
      compiler_params=pltpu.CompilerParams(use_tc_tiling_on_sc=False),
      scratch_types=[
          pltpu.VMEM((NCH, CHUNK), jnp.int32),
          pltpu.VMEM((NCH, CHUNK), jnp.int32),
          pltpu.VMEM((2 * NBUF, CHUNK, D), jnp.float32),
          pltpu.VMEM((ZR, D), jnp.float32),
          pltpu.VMEM_SHARED((NPAD, D), jnp.float32),
          pltpu.SemaphoreType.DMA,
          pltpu.SemaphoreType.DMA,
          pltpu.SemaphoreType.DMA,
          pltpu.SemaphoreType.DMA,
      ],
  )
  return f(sup, src2d, dst2d)


def _mm_xw1(x, W1):
  def body(x_ref, w_ref, o_ref):
    o_ref[...] = jnp.dot(x_ref[...], w_ref[...],
                         preferred_element_type=jnp.float32)

  return pl.pallas_call(
      body,
      grid=(10,),
      in_specs=[pl.BlockSpec((1000, D_IN), lambda i: (i, 0)),
                pl.BlockSpec((D_IN, H1), lambda i: (0, 0))],
      out_specs=pl.BlockSpec((1000, H1), lambda i: (i, 0)),
      out_shape=jax.ShapeDtypeStruct((N, H1), jnp.float32),
  )(x, W1)


def _relu_sum_mm(p1, W2):
  def body(p_ref, w_ref, o_ref):
    h = jnp.maximum(p_ref[0] + p_ref[1], 0.0)
    o_ref[...] = jnp.dot(h, w_ref[...], preferred_element_type=jnp.float32)

  return pl.pallas_call(
      body,
      grid=(10,),
      in_specs=[pl.BlockSpec((NC, 1000, H1), lambda i: (0, i, 0)),
                pl.BlockSpec((H1, H2), lambda i: (0, 0))],
      out_specs=pl.BlockSpec((1000, H2), lambda i: (i, 0)),
      out_shape=jax.ShapeDtypeStruct((N, H2), jnp.float32),
  )(p1, W2)


def _decoder(p2):
  def body(p_blk_ref, p_full_ref, recon_ref, mu_ref):
    zb = p_blk_ref[0] + p_blk_ref[1]
    zf = p_full_ref[0] + p_full_ref[1]
    mu_ref[...] = zb
    recon_ref[...] = lax.dot_general(
        zb.astype(jnp.bfloat16), zf.astype(jnp.bfloat16),
        (((1,), (1,)), ((), ())), preferred_element_type=jnp.float32)

  return pl.pallas_call(
      body,
      grid=(25,),
      in_specs=[pl.BlockSpec((NC, 400, H2), lambda i: (0, i, 0)),
                pl.BlockSpec((NC, N, H2), lambda i: (0, 0, 0))],
      out_specs=[pl.BlockSpec((400, N), lambda i: (i, 0)),
                 pl.BlockSpec((400, H2), lambda i: (i, 0))],
      out_shape=[jax.ShapeDtypeStruct((N, N), jnp.float32),
                 jax.ShapeDtypeStruct((N, H2), jnp.float32)],
  )(p2, p2)


def kernel(x, edge_index, W1, W2):
  src2d = edge_index[0].reshape(NW * NCH, CHUNK)
  dst2d = edge_index[1].reshape(NW * NCH, CHUNK)
  support1 = _mm_xw1(x, W1)
  p1 = _spmm_sc(support1, src2d, dst2d, H1, NBUF=2)
  support2 = _relu_sum_mm(p1, W2)
  p2 = _spmm_sc(support2, src2d, dst2d, H2, NBUF=4)
  recon, mu = _decoder(p2)
  return (recon, mu)


# speedup vs baseline: 1.0097x; 1.0097x over previous
import functools

import jax
import jax.numpy as jnp
from jax import lax
from jax.experimental import pallas as pl
from jax.experimental.pallas import tpu as pltpu
from jax.experimental.pallas import tpu_sc as plsc

N = 10000
E = 320000
D_IN = 128
H1 = 64
H2 = 16

NC = 2
NS = 16
NW = NC * NS
NPAD = 10240
ROWS_PER_TILE = NPAD // NS
ZR = 64


def _spmm_sc(sup, src2d, dst2d, D, NBUF, CHUNK):
  NCH = E // (NW * CHUNK)
  NG = NCH // NBUF
  assert NCH * NW * CHUNK == E and NCH % 8 == 0
  assert NG * NBUF == NCH and NG % 2 == 0
  mesh = plsc.VectorSubcoreMesh(core_axis_name="c", subcore_axis_name="s")

  def body(sup_hbm, src_hbm, dst_hbm, out_hbm,
           src_v, dst_v, rows_v, zblk_v, acc_sh,
           sem_g0, sem_g1, sem_s0, sem_s1):
    cid = lax.axis_index("c")
    sid = lax.axis_index("s")
    w = sid * NC + cid
    sem_g = (sem_g0, sem_g1)
    sem_s = (sem_s0, sem_s1)

    zero16 = jnp.zeros((16,), jnp.float32)

    def zrow(i, carry):
      for c in range(D // 16):
        zblk_v[i, pl.ds(16 * c, 16)] = zero16
      return carry

    lax.fori_loop(0, ZR, zrow, 0)
    for j in range(ROWS_PER_TILE // ZR):
      pltpu.sync_copy(zblk_v,
                      acc_sh.at[pl.ds(sid * ROWS_PER_TILE + j * ZR, ZR), :])
    plsc.subcore_barrier()

    pltpu.sync_copy(src_hbm.at[pl.ds(w * NCH, NCH), :], src_v)
    pltpu.sync_copy(dst_hbm.at[pl.ds(w * NCH, NCH), :], dst_v)

    def fire_g(grp, g):
      for b in range(NBUF):
        pltpu.async_copy(sup_hbm.at[src_v.at[g * NBUF + b]],
                         rows_v.at[grp * NBUF + b], sem_g[grp])

    def drain_g(grp):
      for b in range(NBUF):
        pltpu.make_async_copy(sup_hbm.at[src_v.at[0]],
                              rows_v.at[grp * NBUF + b], sem_g[grp]).wait()

    def fire_s(grp, g):
      for b in range(NBUF):
        pltpu.async_copy(rows_v.at[grp * NBUF + b],
                         acc_sh.at[dst_v.at[g * NBUF + b]], sem_s[grp],
                         add=True)

    def drain_s(grp):
      for b in range(NBUF):
        pltpu.make_async_copy(rows_v.at[grp * NBUF + b],
                              acc_sh.at[dst_v.at[0]], sem_s[grp]).wait()

    fire_g(0, 0)

    def itr(k, carry):
      g0 = 2 * k
      g1 = 2 * k + 1
      drain_g(0)

      @pl.when(k > 0)
      def _():
        drain_s(1)

      fire_g(1, g1)
      fire_s(0, g0)
      drain_g(1)
      drain_s(0)

      @pl.when(k < NG // 2 - 1)
      def _():
        fire_g(0, g0 + 2)

      fire_s(1, g1)
      return carry

    lax.fori_loop(0, NG // 2, itr, 0)
    drain_s(1)
    plsc.subcore_barrier()

    for j in range(ROWS_PER_TILE // ZR):
      r0 = sid * ROWS_PER_TILE + j * ZR
      pltpu.sync_copy(acc_sh.at[pl.ds(r0, ZR), :],
                      out_hbm.at[cid, pl.ds(r0, ZR), :])

  f = pl.kernel(
      body,
      out_type=jax.ShapeDtypeStruct((NC, NPAD, D), jnp.float32),
      mesh=mesh,
      compiler_params=pltpu.CompilerParams(use_tc_tiling_on_sc=False),
      scratch_types=[
          pltpu.VMEM((NCH, CHUNK), jnp.int32),
          pltpu.VMEM((NCH, CHUNK), jnp.int32),
          pltpu.VMEM((2 * NBUF, CHUNK, D), jnp.float32),
          pltpu.VMEM((ZR, D), jnp.float32),
          pltpu.VMEM_SHARED((NPAD, D), jnp.float32),
          pltpu.SemaphoreType.DMA,
          pltpu.SemaphoreType.DMA,
          pltpu.SemaphoreType.DMA,
          pltpu.SemaphoreType.DMA,
      ],
  )
  return f(sup, src2d, dst2d)


def _mm_xw1(x, W1):
  def body(x_ref, w_ref, o_ref):
    o_ref[...] = jnp.dot(x_ref[...], w_ref[...],
                         preferred_element_type=jnp.float32)

  return pl.pallas_call(
      body,
      grid=(10,),
      in_specs=[pl.BlockSpec((1000, D_IN), lambda i: (i, 0)),
                pl.BlockSpec((D_IN, H1), lambda i: (0, 0))],
      out_specs=pl.BlockSpec((1000, H1), lambda i: (i, 0)),
      out_shape=jax.ShapeDtypeStruct((N, H1), jnp.float32),
  )(x, W1)


def _relu_sum_mm(p1, W2):
  def body(p_ref, w_ref, o_ref):
    h = jnp.maximum(p_ref[0] + p_ref[1], 0.0)
    o_ref[...] = jnp.dot(h, w_ref[...], preferred_element_type=jnp.float32)

  return pl.pallas_call(
      body,
      grid=(10,),
      in_specs=[pl.BlockSpec((NC, 1000, H1), lambda i: (0, i, 0)),
                pl.BlockSpec((H1, H2), lambda i: (0, 0))],
      out_specs=pl.BlockSpec((1000, H2), lambda i: (i, 0)),
      out_shape=jax.ShapeDtypeStruct((N, H2), jnp.float32),
  )(p1, W2)


def _decoder(p2):
  def body(p_blk_ref, p_full_ref, recon_ref, mu_ref):
    zb = p_blk_ref[0] + p_blk_ref[1]
    zf = p_full_ref[0] + p_full_ref[1]
    mu_ref[...] = zb
    recon_ref[...] = lax.dot_general(
        zb.astype(jnp.bfloat16), zf.astype(jnp.bfloat16),
        (((1,), (1,)), ((), ())), preferred_element_type=jnp.float32)

  return pl.pallas_call(
      body,
      grid=(25,),
      in_specs=[pl.BlockSpec((NC, 400, H2), lambda i: (0, i, 0)),
                pl.BlockSpec((NC, N, H2), lambda i: (0, 0, 0))],
      out_specs=[pl.BlockSpec((400, N), lambda i: (i, 0)),
                 pl.BlockSpec((400, H2), lambda i: (i, 0))],
      out_shape=[jax.ShapeDtypeStruct((N, N), jnp.float32),
                 jax.ShapeDtypeStruct((N, H2), jnp.float32)],
  )(p2, p2)


def kernel(x, edge_index, W1, W2):
  src = edge_index[0]
  dst = edge_index[1]
  support1 = _mm_xw1(x, W1)
  p1 = _spmm_sc(support1, src.reshape(-1, 125), dst.reshape(-1, 125),
                H1, NBUF=4, CHUNK=125)
  support2 = _relu_sum_mm(p1, W2)
  p2 = _spmm_sc(support2, src.reshape(-1, 250), dst.reshape(-1, 250),
                H2, NBUF=5, CHUNK=250)
  recon, mu = _decoder(p2)
  return (recon, mu)


# speedup vs baseline: 1.0113x; 1.0016x over previous
import functools

import jax
import jax.numpy as jnp
from jax import lax
from jax.experimental import pallas as pl
from jax.experimental.pallas import tpu as pltpu
from jax.experimental.pallas import tpu_sc as plsc

N = 10000
E = 320000
D_IN = 128
H1 = 64
H2 = 16

NC = 2
NS = 16
NW = NC * NS
NPAD = 10240
ROWS_PER_TILE = NPAD // NS
ZR = 64


def _spmm_sc(sup, src2d, dst2d, D, NBUF, CHUNK):
  NCH = E // (NW * CHUNK)
  NG = NCH // NBUF
  assert NCH * NW * CHUNK == E and NCH % 8 == 0
  assert NG * NBUF == NCH and NG % 2 == 0
  mesh = plsc.VectorSubcoreMesh(core_axis_name="c", subcore_axis_name="s")

  def body(sup_hbm, src_hbm, dst_hbm, out_hbm,
           src_v, dst_v, rows_v, zblk_v, acc_sh,
           sem_g0, sem_g1, sem_s0, sem_s1):
    cid = lax.axis_index("c")
    sid = lax.axis_index("s")
    w = sid * NC + cid
    sem_g = (sem_g0, sem_g1)
    sem_s = (sem_s0, sem_s1)

    zero16 = jnp.zeros((16,), jnp.float32)

    def zrow(i, carry):
      for c in range(D // 16):
        zblk_v[i, pl.ds(16 * c, 16)] = zero16
      return carry

    lax.fori_loop(0, ZR, zrow, 0)
    for j in range(ROWS_PER_TILE // ZR):
      pltpu.sync_copy(zblk_v,
                      acc_sh.at[pl.ds(sid * ROWS_PER_TILE + j * ZR, ZR), :])
    plsc.subcore_barrier()

    pltpu.sync_copy(src_hbm.at[pl.ds(w * NCH, NCH), :], src_v)
    pltpu.sync_copy(dst_hbm.at[pl.ds(w * NCH, NCH), :], dst_v)

    def fire_g(grp, g):
      for b in range(NBUF):
        pltpu.async_copy(sup_hbm.at[src_v.at[g * NBUF + b]],
                         rows_v.at[grp * NBUF + b], sem_g[grp])

    def drain_g(grp):
      for b in range(NBUF):
        pltpu.make_async_copy(sup_hbm.at[src_v.at[0]],
                              rows_v.at[grp * NBUF + b], sem_g[grp]).wait()

    def fire_s(grp, g):
      for b in range(NBUF):
        pltpu.async_copy(rows_v.at[grp * NBUF + b],
                         acc_sh.at[dst_v.at[g * NBUF + b]], sem_s[grp],
                         add=True)

    def drain_s(grp):
      for b in range(NBUF):
        pltpu.make_async_copy(rows_v.at[grp * NBUF + b],
                              acc_sh.at[dst_v.at[0]], sem_s[grp]).wait()

    fire_g(0, 0)

    def itr(k, carry):
      g0 = 2 * k
      g1 = 2 * k + 1
      drain_g(0)

      @pl.when(k > 0)
      def _():
        drain_s(1)

      fire_g(1, g1)
      fire_s(0, g0)
      drain_g(1)
      drain_s(0)

      @pl.when(k < NG // 2 - 1)
      def _():
        fire_g(0, g0 + 2)

      fire_s(1, g1)
      return carry

    lax.fori_loop(0, NG // 2, itr, 0)
    drain_s(1)
    plsc.subcore_barrier()

    for j in range(ROWS_PER_TILE // ZR):
      r0 = sid * ROWS_PER_TILE + j * ZR
      pltpu.sync_copy(acc_sh.at[pl.ds(r0, ZR), :],
                      out_hbm.at[cid, pl.ds(r0, ZR), :])

  f = pl.kernel(
      body,
      out_type=jax.ShapeDtypeStruct((NC, NPAD, D), jnp.float32),
      mesh=mesh,
      compiler_params=pltpu.CompilerParams(use_tc_tiling_on_sc=False),
      scratch_types=[
          pltpu.VMEM((NCH, CHUNK), jnp.int32),
          pltpu.VMEM((NCH, CHUNK), jnp.int32),
          pltpu.VMEM((2 * NBUF, CHUNK, D), jnp.float32),
          pltpu.VMEM((ZR, D), jnp.float32),
          pltpu.VMEM_SHARED((NPAD, D), jnp.float32),
          pltpu.SemaphoreType.DMA,
          pltpu.SemaphoreType.DMA,
          pltpu.SemaphoreType.DMA,
          pltpu.SemaphoreType.DMA,
      ],
  )
  return f(sup, src2d, dst2d)


def _mm_xw1(x, W1):
  def body(x_ref, w_ref, o_ref):
    o_ref[...] = jnp.dot(x_ref[...], w_ref[...],
                         preferred_element_type=jnp.float32)

  return pl.pallas_call(
      body,
      grid=(10,),
      in_specs=[pl.BlockSpec((1000, D_IN), lambda i: (i, 0)),
                pl.BlockSpec((D_IN, H1), lambda i: (0, 0))],
      out_specs=pl.BlockSpec((1000, H1), lambda i: (i, 0)),
      out_shape=jax.ShapeDtypeStruct((N, H1), jnp.float32),
  )(x, W1)


def _relu_sum_mm(p1, W2):
  def body(p_ref, w_ref, o_ref):
    h = jnp.maximum(p_ref[0] + p_ref[1], 0.0)
    o_ref[...] = jnp.dot(h, w_ref[...], preferred_element_type=jnp.float32)

  return pl.pallas_call(
      body,
      grid=(10,),
      in_specs=[pl.BlockSpec((NC, 1000, H1), lambda i: (0, i, 0)),
                pl.BlockSpec((H1, H2), lambda i: (0, 0))],
      out_specs=pl.BlockSpec((1000, H2), lambda i: (i, 0)),
      out_shape=jax.ShapeDtypeStruct((N, H2), jnp.float32),
  )(p1, W2)


def _decoder(p2):
  def body(p_blk_ref, p_full_ref, recon_ref, mu_ref):
    zb = p_blk_ref[0] + p_blk_ref[1]
    zf = p_full_ref[0] + p_full_ref[1]
    mu_ref[...] = zb
    recon_ref[...] = lax.dot_general(
        zb.astype(jnp.bfloat16), zf.astype(jnp.bfloat16),
        (((1,), (1,)), ((), ())), preferred_element_type=jnp.float32)

  return pl.pallas_call(
      body,
      grid=(50,),
      in_specs=[pl.BlockSpec((NC, 200, H2), lambda i: (0, i, 0)),
                pl.BlockSpec((NC, N, H2), lambda i: (0, 0, 0))],
      out_specs=[pl.BlockSpec((200, N), lambda i: (i, 0)),
                 pl.BlockSpec((200, H2), lambda i: (i, 0))],
      out_shape=[jax.ShapeDtypeStruct((N, N), jnp.float32),
                 jax.ShapeDtypeStruct((N, H2), jnp.float32)],
  )(p2, p2)


def kernel(x, edge_index, W1, W2):
  src = edge_index[0]
  dst = edge_index[1]
  support1 = _mm_xw1(x, W1)
  p1 = _spmm_sc(support1, src.reshape(-1, 125), dst.reshape(-1, 125),
                H1, NBUF=4, CHUNK=125)
  support2 = _relu_sum_mm(p1, W2)
  p2 = _spmm_sc(support2, src.reshape(-1, 250), dst.reshape(-1, 250),
                H2, NBUF=5, CHUNK=250)
  recon, mu = _decoder(p2)
  return (recon, mu)


# speedup vs baseline: 1.0114x; 1.0001x over previous
import functools

import jax
import jax.numpy as jnp
from jax import lax
from jax.experimental import pallas as pl
from jax.experimental.pallas import tpu as pltpu
from jax.experimental.pallas import tpu_sc as plsc

N = 10000
E = 320000
D_IN = 128
H1 = 64
H2 = 16

NC = 2
NS = 16
NW = NC * NS
NPAD = 10240
ROWS_PER_TILE = NPAD // NS
ZR = 64


def _spmm_sc(sup, src2d, dst2d, D, NBUF, CHUNK):
  NCH = E // (NW * CHUNK)
  NG = NCH // NBUF
  assert NCH * NW * CHUNK == E and NCH % 8 == 0
  assert NG * NBUF == NCH and NG % 2 == 0
  mesh = plsc.VectorSubcoreMesh(core_axis_name="c", subcore_axis_name="s")

  def body(sup_hbm, src_hbm, dst_hbm, out_hbm,
           src_v, dst_v, rows_v, zblk_v, acc_sh,
           sem_g0, sem_g1, sem_s0, sem_s1):
    cid = lax.axis_index("c")
    sid = lax.axis_index("s")
    w = sid * NC + cid
    sem_g = (sem_g0, sem_g1)
    sem_s = (sem_s0, sem_s1)

    zero16 = jnp.zeros((16,), jnp.float32)

    def zrow(i, carry):
      for c in range(D // 16):
        zblk_v[i, pl.ds(16 * c, 16)] = zero16
      return carry

    lax.fori_loop(0, ZR, zrow, 0)
    for j in range(ROWS_PER_TILE // ZR):
      pltpu.sync_copy(zblk_v,
                      acc_sh.at[pl.ds(sid * ROWS_PER_TILE + j * ZR, ZR), :])
    plsc.subcore_barrier()

    pltpu.sync_copy(src_hbm.at[pl.ds(w * NCH, NCH), :], src_v)
    pltpu.sync_copy(dst_hbm.at[pl.ds(w * NCH, NCH), :], dst_v)

    def fire_g(grp, g):
      for b in range(NBUF):
        pltpu.async_copy(sup_hbm.at[src_v.at[g * NBUF + b]],
                         rows_v.at[grp * NBUF + b], sem_g[grp])

    def drain_g(grp):
      for b in range(NBUF):
        pltpu.make_async_copy(sup_hbm.at[src_v.at[0]],
                              rows_v.at[grp * NBUF + b], sem_g[grp]).wait()

    def fire_s(grp, g):
      for b in range(NBUF):
        pltpu.async_copy(rows_v.at[grp * NBUF + b],
                         acc_sh.at[dst_v.at[g * NBUF + b]], sem_s[grp],
                         add=True)

    def drain_s(grp):
      for b in range(NBUF):
        pltpu.make_async_copy(rows_v.at[grp * NBUF + b],
                              acc_sh.at[dst_v.at[0]], sem_s[grp]).wait()

    fire_g(0, 0)

    def itr(k, carry):
      g0 = 2 * k
      g1 = 2 * k + 1
      drain_g(0)

      @pl.when(k > 0)
      def _():
        drain_s(1)

      fire_g(1, g1)
      fire_s(0, g0)
      drain_g(1)
      drain_s(0)

      @pl.when(k < NG // 2 - 1)
      def _():
        fire_g(0, g0 + 2)

      fire_s(1, g1)
      return carry

    lax.fori_loop(0, NG // 2, itr, 0)
    drain_s(1)
    plsc.subcore_barrier()

    for j in range(ROWS_PER_TILE // ZR):
      r0 = sid * ROWS_PER_TILE + j * ZR
      pltpu.sync_copy(acc_sh.at[pl.ds(r0, ZR), :],
                      out_hbm.at[cid, pl.ds(r0, ZR), :])

  f = pl.kernel(
      body,
      out_type=jax.ShapeDtypeStruct((NC, NPAD, D), jnp.float32),
      mesh=mesh,
      compiler_params=pltpu.CompilerParams(use_tc_tiling_on_sc=False),
      scratch_types=[
          pltpu.VMEM((NCH, CHUNK), jnp.int32),
          pltpu.VMEM((NCH, CHUNK), jnp.int32),
          pltpu.VMEM((2 * NBUF, CHUNK, D), jnp.float32),
          pltpu.VMEM((ZR, D), jnp.float32),
          pltpu.VMEM_SHARED((NPAD, D), jnp.float32),
          pltpu.SemaphoreType.DMA,
          pltpu.SemaphoreType.DMA,
          pltpu.SemaphoreType.DMA,
          pltpu.SemaphoreType.DMA,
      ],
  )
  return f(sup, src2d, dst2d)


def _mm_xw1(x, W1):
  def body(x_ref, w_ref, o_ref):
    o_ref[...] = jnp.dot(x_ref[...], w_ref[...],
                         preferred_element_type=jnp.float32)

  return pl.pallas_call(
      body,
      grid=(10,),
      in_specs=[pl.BlockSpec((1000, D_IN), lambda i: (i, 0)),
                pl.BlockSpec((D_IN, H1), lambda i: (0, 0))],
      out_specs=pl.BlockSpec((1000, H1), lambda i: (i, 0)),
      out_shape=jax.ShapeDtypeStruct((N, H1), jnp.float32),
  )(x, W1)


def _relu_sum_mm(p1, W2):
  def body(p_ref, w_ref, o_ref):
    h = jnp.maximum(p_ref[0] + p_ref[1], 0.0)
    o_ref[...] = jnp.dot(h, w_ref[...], preferred_element_type=jnp.float32)

  return pl.pallas_call(
      body,
      grid=(10,),
      in_specs=[pl.BlockSpec((NC, 1000, H1), lambda i: (0, i, 0)),
                pl.BlockSpec((H1, H2), lambda i: (0, 0))],
      out_specs=pl.BlockSpec((1000, H2), lambda i: (i, 0)),
      out_shape=jax.ShapeDtypeStruct((N, H2), jnp.float32),
  )(p1, W2)


def _decoder(p2):
  def body(p_blk_ref, p_full_ref, recon_ref, mu_ref):
    zb = p_blk_ref[0] + p_blk_ref[1]
    zf = p_full_ref[0] + p_full_ref[1]
    mu_ref[...] = zb
    recon_ref[...] = lax.dot_general(
        zb.astype(jnp.bfloat16), zf.astype(jnp.bfloat16),
        (((1,), (1,)), ((), ())), preferred_element_type=jnp.float32)

  return pl.pallas_call(
      body,
      grid=(25,),
      in_specs=[pl.BlockSpec((NC, 400, H2), lambda i: (0, i, 0)),
                pl.BlockSpec((NC, N, H2), lambda i: (0, 0, 0))],
      out_specs=[pl.BlockSpec((400, N), lambda i: (i, 0)),
                 pl.BlockSpec((400, H2), lambda i: (i, 0))],
      out_shape=[jax.ShapeDtypeStruct((N, N), jnp.float32),
                 jax.ShapeDtypeStruct((N, H2), jnp.float32)],
  )(p2, p2)


def kernel(x, edge_index, W1, W2):
  src = edge_index[0]
  dst = edge_index[1]
  support1 = _mm_xw1(x, W1)
  p1 = _spmm_sc(support1, src.reshape(-1, 125), dst.reshape(-1, 125),
                H1, NBUF=4, CHUNK=125)
  support2 = _relu_sum_mm(p1, W2)
  p2 = _spmm_sc(support2, src.reshape(-1, 625), dst.reshape(-1, 625),
                H2, NBUF=2, CHUNK=625)
  recon, mu = _decoder(p2)
  return (recon, mu)


# speedup vs baseline: 1.0117x; 1.0004x over previous
import jax
import jax.numpy as jnp
from jax import lax
from jax.experimental import pallas as pl
from jax.experimental.pallas import tpu as pltpu
from jax.experimental.pallas import tpu_sc as plsc

N = 10000
E = 320000
D_IN = 128
H1 = 64
H2 = 16

NC = 2
NS = 16
NW = NC * NS
NPAD = 10240
ROWS_PER_TILE = NPAD // NS
ZR = 64


def _spmm_sc(sup, src2d, dst2d, D, NBUF, CHUNK):
  NCH = E // (NW * CHUNK)
  NG = NCH // NBUF
  assert NCH * NW * CHUNK == E and NCH % 8 == 0
  assert NG * NBUF == NCH and NG % 2 == 0
  mesh = plsc.VectorSubcoreMesh(core_axis_name="c", subcore_axis_name="s")

  def body(sup_hbm, src_hbm, dst_hbm, out_hbm,
           src_v, dst_v, rows_v, zblk_v, acc_sh,
           sem_g0, sem_g1, sem_s0, sem_s1):
    cid = lax.axis_index("c")
    sid = lax.axis_index("s")
    w = sid * NC + cid
    sem_g = (sem_g0, sem_g1)
    sem_s = (sem_s0, sem_s1)

    zero16 = jnp.zeros((16,), jnp.float32)

    def zrow(i, carry):
      for c in range(D // 16):
        zblk_v[i, pl.ds(16 * c, 16)] = zero16
      return carry

    lax.fori_loop(0, ZR, zrow, 0)
    for j in range(ROWS_PER_TILE // ZR):
      pltpu.sync_copy(zblk_v,
                      acc_sh.at[pl.ds(sid * ROWS_PER_TILE + j * ZR, ZR), :])
    plsc.subcore_barrier()

    pltpu.sync_copy(src_hbm.at[pl.ds(w * NCH, NCH), :], src_v)
    pltpu.sync_copy(dst_hbm.at[pl.ds(w * NCH, NCH), :], dst_v)

    def fire_g(grp, g):
      for b in range(NBUF):
        pltpu.async_copy(sup_hbm.at[src_v.at[g * NBUF + b]],
                         rows_v.at[grp * NBUF + b], sem_g[grp])

    def drain_g(grp):
      for b in range(NBUF):
        pltpu.make_async_copy(sup_hbm.at[src_v.at[0]],
                              rows_v.at[grp * NBUF + b], sem_g[grp]).wait()

    def fire_s(grp, g):
      for b in range(NBUF):
        pltpu.async_copy(rows_v.at[grp * NBUF + b],
                         acc_sh.at[dst_v.at[g * NBUF + b]], sem_s[grp],
                         add=True)

    def drain_s(grp):
      for b in range(NBUF):
        pltpu.make_async_copy(rows_v.at[grp * NBUF + b],
                              acc_sh.at[dst_v.at[0]], sem_s[grp]).wait()

    fire_g(0, 0)

    def itr(k, carry):
      g0 = 2 * k
      g1 = 2 * k + 1
      drain_g(0)

      @pl.when(k > 0)
      def _():
        drain_s(1)

      fire_g(1, g1)
      fire_s(0, g0)
      drain_g(1)
      drain_s(0)

      @pl.when(k < NG // 2 - 1)
      def _():
        fire_g(0, g0 + 2)

      fire_s(1, g1)
      return carry

    lax.fori_loop(0, NG // 2, itr, 0)
    drain_s(1)
    plsc.subcore_barrier()

    for j in range(ROWS_PER_TILE // ZR):
      r0 = sid * ROWS_PER_TILE + j * ZR
      pltpu.sync_copy(acc_sh.at[pl.ds(r0, ZR), :],
                      out_hbm.at[cid, pl.ds(r0, ZR), :])

  f = pl.kernel(
      body,
      out_type=jax.ShapeDtypeStruct((NC, NPAD, D), jnp.float32),
      mesh=mesh,
      compiler_params=pltpu.CompilerParams(use_tc_tiling_on_sc=False),
      scratch_types=[
          pltpu.VMEM((NCH, CHUNK), jnp.int32),
          pltpu.VMEM((NCH, CHUNK), jnp.int32),
          pltpu.VMEM((2 * NBUF, CHUNK, D), jnp.float32),
          pltpu.VMEM((ZR, D), jnp.float32),
          pltpu.VMEM_SHARED((NPAD, D), jnp.float32),
          pltpu.SemaphoreType.DMA,
          pltpu.SemaphoreType.DMA,
          pltpu.SemaphoreType.DMA,
          pltpu.SemaphoreType.DMA,
      ],
  )
  return f(sup, src2d, dst2d)


def _mm_xw1(x, W1):
  def body(x_ref, w_ref, o_ref):
    o_ref[...] = jnp.dot(x_ref[...], w_ref[...],
                         preferred_element_type=jnp.float32)

  return pl.pallas_call(
      body,
      grid=(10,),
      in_specs=[pl.BlockSpec((1000, D_IN), lambda i: (i, 0)),
                pl.BlockSpec((D_IN, H1), lambda i: (0, 0))],
      out_specs=pl.BlockSpec((1000, H1), lambda i: (i, 0)),
      out_shape=jax.ShapeDtypeStruct((N, H1), jnp.float32),
  )(x, W1)


def _relu_sum_mm(p1, W2):
  def body(p_ref, w_ref, o_ref):
    h = jnp.maximum(p_ref[0] + p_ref[1], 0.0)
    o_ref[...] = jnp.dot(h, w_ref[...], preferred_element_type=jnp.float32)

  return pl.pallas_call(
      body,
      grid=(10,),
      in_specs=[pl.BlockSpec((NC, 1000, H1), lambda i: (0, i, 0)),
                pl.BlockSpec((H1, H2), lambda i: (0, 0))],
      out_specs=pl.BlockSpec((1000, H2), lambda i: (i, 0)),
      out_shape=jax.ShapeDtypeStruct((N, H2), jnp.float32),
  )(p1, W2)


def _decoder(p2):
  def body(p_blk_ref, p_full_ref, recon_ref, mu_ref):
    zb = p_blk_ref[0] + p_blk_ref[1]
    zf = p_full_ref[0] + p_full_ref[1]
    mu_ref[...] = zb
    recon_ref[...] = lax.dot_general(
        zb.astype(jnp.bfloat16), zf.astype(jnp.bfloat16),
        (((1,), (1,)), ((), ())), preferred_element_type=jnp.float32)

  return pl.pallas_call(
      body,
      grid=(25,),
      in_specs=[pl.BlockSpec((NC, 400, H2), lambda i: (0, i, 0)),
                pl.BlockSpec((NC, N, H2), lambda i: (0, 0, 0))],
      out_specs=[pl.BlockSpec((400, N), lambda i: (i, 0)),
                 pl.BlockSpec((400, H2), lambda i: (i, 0))],
      out_shape=[jax.ShapeDtypeStruct((N, N), jnp.float32),
                 jax.ShapeDtypeStruct((N, H2), jnp.float32)],
  )(p2, p2)


def kernel(x, edge_index, W1, W2):
  src = edge_index[0]
  dst = edge_index[1]
  support1 = _mm_xw1(x, W1)
  p1 = _spmm_sc(support1, src.reshape(-1, 125), dst.reshape(-1, 125),
                H1, NBUF=4, CHUNK=125)
  support2 = _relu_sum_mm(p1, W2)
  p2 = _spmm_sc(support2, src.reshape(-1, 625), dst.reshape(-1, 625),
                H2, NBUF=2, CHUNK=625)
  recon, mu = _decoder(p2)
  return (recon, mu)
